# baseline (speedup 1.0000x reference)
import jax
import jax.numpy as jnp
from jax.experimental import pallas as pl
from jax.experimental.pallas import tpu as pltpu

_D = 16
_HQ = 16
_CQ = 4


_TN = (((0,), (0,)), ((), ()))


def _mlp_t_kernel(xt_ref, w1_ref, b1_ref, w3_ref, b3_ref, o_ref):
    w1s = w1_ref[:, :_HQ]
    w3s = w3_ref[:_HQ, :_CQ]
    b1c = b1_ref[:, :_HQ].T
    b3c = b3_ref[:, :_CQ].T
    h = jax.lax.dot_general(w1s, xt_ref[...], _TN, preferred_element_type=jnp.float32)
    h = jnp.maximum(h + b1c, 0.0)
    out = jax.lax.dot_general(w3s, h, _TN, preferred_element_type=jnp.float32)
    o_ref[...] = out + b3c


def kernel(x, w1_t, b1_p, w3_t, b3_p):
    B = x.shape[0]
    C = _CQ

    xt = x.T

    TBc = min(262144, B)
    grid = (pl.cdiv(B, TBc),)

    outT = pl.pallas_call(
        _mlp_t_kernel,
        out_shape=jax.ShapeDtypeStruct((C, B), jnp.float32),
        grid=grid,
        in_specs=[
            pl.BlockSpec((_D, TBc), lambda i: (0, i)),
            pl.BlockSpec((_D, 128), lambda i: (0, 0)),
            pl.BlockSpec((1, 128), lambda i: (0, 0)),
            pl.BlockSpec((128, 128), lambda i: (0, 0)),
            pl.BlockSpec((1, 128), lambda i: (0, 0)),
        ],
        out_specs=pl.BlockSpec((C, TBc), lambda i: (0, i)),
        compiler_params=pltpu.CompilerParams(
            dimension_semantics=("parallel",),
            vmem_limit_bytes=96 << 20,
        ),
        cost_estimate=pl.CostEstimate(
            flops=2 * B * (_D * _HQ + _HQ * C),
            transcendentals=0,
            bytes_accessed=4 * (B * _D + B * C),
        ),
    )(xt, w1_t, b1_p, w3_t, b3_p)

    return outT.T
